# baseline (speedup 1.0000x reference)
import jax
import jax.numpy as jnp
from jax import lax
from jax.experimental import pallas as pl
from jax.experimental.pallas import tpu as pltpu
from jax.experimental.pallas import tpu_sc as plsc

N = 32768
K = 25
NS = 16
L = 16
CHUNK = N // NS
VPC = CHUNK // L
NBINS = 256
NBV = NBINS // L
ROWS = 1024


def _sc_topk_body(x_hbm, out_hbm, x_v, key_v, hist_v, mhist_v,
                  idx_v, eqi_v, eq_v, out_v, tmp_v, sem, sem2, sh_flat):
    sid = lax.axis_index("s")
    base = sid * CHUNK
    cp = pltpu.async_copy(x_hbm.at[pl.ds(base, CHUNK)], x_v, sem)

    iota = lax.iota(jnp.int32, L)
    zeros_i = jnp.zeros((L,), jnp.int32)
    ones_i = jnp.ones((L,), jnp.int32)

    for g in range(8):
        for t in range(8):
            idx_v[g, pl.ds(t * L, L)] = iota + (g * 128 + t * L)
    for t in range(4):
        tmp_v[pl.ds(t * L, L)] = zeros_i
    for t in range(NBV):
        hist_v[pl.ds(t * L, L)] = zeros_i
    pltpu.sync_copy(tmp_v.at[pl.ds(0, 64)], sh_flat.at[pl.ds(sid * 64, 64)])
    cp.wait()
    plsc.subcore_barrier()

    pref = jnp.zeros((L,), jnp.uint32)
    rem = jnp.full((L,), K, jnp.int32)

    def select_digit(rem):
        totals = zeros_i
        for t in range(NBV):
            h = mhist_v[pl.ds(t * L, L)]
            totals = totals + jnp.where(iota == t, jnp.sum(h), 0)
        rts = lax.rev(plsc.cumsum(lax.rev(totals, (0,))), (0,))
        tstar = plsc.all_reduce_population_count(rts >= rem) - 1
        excl = jnp.sum(jnp.where(iota == tstar, rts - totals, 0))
        ts = jnp.max(tstar)
        h = mhist_v[pl.ds(ts * L, L)]
        suf = lax.rev(plsc.cumsum(lax.rev(h, (0,))), (0,)) + excl
        cnt_in = plsc.all_reduce_population_count(suf >= rem)
        bstar = tstar * L + cnt_in - 1
        lane = cnt_in - 1
        suf_b = jnp.sum(jnp.where(iota == lane, suf, 0))
        h_b = jnp.sum(jnp.where(iota == lane, h, 0))
        return bstar, rem - (suf_b - h_b)

    def merge_and_read(r):
        with jax.named_scope("merge%d" % r):
            c1 = pltpu.async_copy(hist_v.at[pl.ds(0, 128)],
                                  sh_flat.at[idx_v.at[2 * r]], sem, add=True)
            c2 = pltpu.async_copy(hist_v.at[pl.ds(128, 128)],
                                  sh_flat.at[idx_v.at[2 * r + 1]], sem2,
                                  add=True)
            c1.wait()
            c2.wait()
        with jax.named_scope("barrier%d" % r):
            plsc.subcore_barrier()
        with jax.named_scope("read%d" % r):
            pltpu.sync_copy(sh_flat.at[pl.ds(r * NBINS, NBINS)], mhist_v)

    with jax.named_scope("pass0"):
        @plsc.parallel_loop(0, VPC, unroll=16)
        def _(j):
            b = lax.bitcast_convert_type(x_v[pl.ds(j * L, L)], jnp.uint32)
            neg = (b >> 31) == jnp.uint32(1)
            key = jnp.where(neg, ~b, b | jnp.uint32(0x80000000))
            key_v[pl.ds(j * L, L)] = key
            digit = (key >> jnp.uint32(24)).astype(jnp.int32)
            plsc.addupdate_scatter(hist_v, [digit], ones_i)

    merge_and_read(0)
    with jax.named_scope("scan0"):
        bstar, rem = select_digit(rem)
        pref = bstar.astype(jnp.uint32) << jnp.uint32(24)

    for r in (1, 2, 3):
        shift = 24 - 8 * r
        hi = jnp.uint32(32 - 8 * r)
        for t in range(NBV):
            hist_v[pl.ds(t * L, L)] = zeros_i
        ph = pref >> hi

        with jax.named_scope("passN"):
            @plsc.parallel_loop(0, VPC, unroll=16)
            def _(j, hi=hi, shift=shift, ph=ph):
                key = key_v[pl.ds(j * L, L)]
                active = (key >> hi) == ph
                digit = ((key >> jnp.uint32(shift)) & jnp.uint32(0xFF))
                plsc.addupdate_scatter(hist_v, [digit.astype(jnp.int32)],
                                       ones_i, mask=active)

        if r == 3:
            pltpu.sync_copy(hist_v,
                            sh_flat.at[pl.ds(ROWS + sid * NBINS, NBINS)])
        merge_and_read(r)
        with jax.named_scope("scan%d" % r):
            bstar, rem = select_digit(rem)
            pref = pref | (bstar.astype(jnp.uint32) << jnp.uint32(shift))

    thresh = pref

    eqi_v[...] = ROWS + iota * NBINS + bstar
    pltpu.sync_copy(sh_flat.at[eqi_v], eq_v)
    ecnt = eq_v[...]
    exclv = plsc.cumsum(ecnt) - ecnt
    ecarry = jnp.sum(jnp.where(iota == sid, exclv, 0))

    with jax.named_scope("final"):
        @plsc.parallel_loop(0, VPC, unroll=16, carry=zeros_i)
        def _(j, qcarry):
            key = key_v[pl.ds(j * L, L)]
            x = x_v[pl.ds(j * L, L)]
            eq = key == thresh
            eqi = eq.astype(jnp.int32)
            incl = plsc.cumsum(eqi)
            rank = incl - eqi + qcarry + ecarry
            keep = (key > thresh) | (eq & (rank < rem))
            out_v[pl.ds(j * L, L)] = jnp.where(keep, x, jnp.float32(0.0))
            return qcarry + plsc.all_reduce_population_count(eq)

    with jax.named_scope("outdma"):
        pltpu.sync_copy(out_v, out_hbm.at[pl.ds(base, CHUNK)])


@jax.jit
def _topk_mask_sc(x_flat):
    mesh = plsc.VectorSubcoreMesh(core_axis_name="c", subcore_axis_name="s",
                                  num_cores=1, num_subcores=NS)
    return pl.kernel(
        _sc_topk_body,
        out_type=jax.ShapeDtypeStruct((N,), jnp.float32),
        mesh=mesh,
        compiler_params=pltpu.CompilerParams(needs_layout_passes=False),
        scratch_types=[
            pltpu.VMEM((CHUNK,), jnp.float32),
            pltpu.VMEM((CHUNK,), jnp.uint32),
            pltpu.VMEM((NBINS,), jnp.int32),
            pltpu.VMEM((NBINS,), jnp.int32),
            pltpu.VMEM((8, 128), jnp.int32),
            pltpu.VMEM((L,), jnp.int32),
            pltpu.VMEM((L,), jnp.int32),
            pltpu.VMEM((CHUNK,), jnp.float32),
            pltpu.VMEM((128,), jnp.int32),
            pltpu.SemaphoreType.DMA,
            pltpu.SemaphoreType.DMA,
            pltpu.VMEM_SHARED((5120,), jnp.int32),
        ],
    )(x_flat)


def kernel(score_vector):
    return _topk_mask_sc(score_vector.reshape(N)).reshape(1, N)


# speedup vs baseline: 1.0639x; 1.0639x over previous
import jax
import jax.numpy as jnp
from jax import lax
from jax.experimental import pallas as pl
from jax.experimental.pallas import tpu as pltpu
from jax.experimental.pallas import tpu_sc as plsc

N = 32768
K = 25
NS = 16
L = 16
CHUNK = N // NS
VPC = CHUNK // L
NBINS = 256
NBV = NBINS // L
ROWS = 1024


def _sc_topk_body(x_hbm, out_hbm, x_v, key_v, hist_v, mhist_v,
                  idx_v, eqi_v, eq_v, out_v, tmp_v, sem, sem2, sh_flat):
    sid = lax.axis_index("s")
    base = sid * CHUNK
    cp = pltpu.async_copy(x_hbm.at[pl.ds(base, CHUNK)], x_v, sem)

    iota = lax.iota(jnp.int32, L)
    zeros_i = jnp.zeros((L,), jnp.int32)
    ones_i = jnp.ones((L,), jnp.int32)

    for g in range(8):
        for t in range(8):
            idx_v[g, pl.ds(t * L, L)] = iota + (g * 128 + t * L)
    for t in range(4):
        tmp_v[pl.ds(t * L, L)] = zeros_i
    for t in range(NBV):
        hist_v[pl.ds(t * L, L)] = zeros_i
    pltpu.sync_copy(tmp_v.at[pl.ds(0, 64)], sh_flat.at[pl.ds(sid * 64, 64)])
    cp.wait()
    plsc.subcore_barrier()

    pref = jnp.zeros((L,), jnp.uint32)
    rem = jnp.full((L,), K, jnp.int32)

    def select_digit(rem):
        totals = zeros_i
        for t in range(NBV):
            h = mhist_v[pl.ds(t * L, L)]
            totals = totals + jnp.where(iota == t, jnp.sum(h), 0)
        rts = lax.rev(plsc.cumsum(lax.rev(totals, (0,))), (0,))
        tstar = plsc.all_reduce_population_count(rts >= rem) - 1
        excl = jnp.sum(jnp.where(iota == tstar, rts - totals, 0))
        ts = jnp.max(tstar)
        h = mhist_v[pl.ds(ts * L, L)]
        suf = lax.rev(plsc.cumsum(lax.rev(h, (0,))), (0,)) + excl
        cnt_in = plsc.all_reduce_population_count(suf >= rem)
        bstar = tstar * L + cnt_in - 1
        lane = cnt_in - 1
        suf_b = jnp.sum(jnp.where(iota == lane, suf, 0))
        h_b = jnp.sum(jnp.where(iota == lane, h, 0))
        return bstar, rem - (suf_b - h_b)

    def merge_and_read(r):
        with jax.named_scope("merge%d" % r):
            c1 = pltpu.async_copy(hist_v.at[pl.ds(0, 128)],
                                  sh_flat.at[idx_v.at[2 * r]], sem, add=True)
            c2 = pltpu.async_copy(hist_v.at[pl.ds(128, 128)],
                                  sh_flat.at[idx_v.at[2 * r + 1]], sem2,
                                  add=True)
            c1.wait()
            c2.wait()
        with jax.named_scope("barrier%d" % r):
            plsc.subcore_barrier()
        with jax.named_scope("read%d" % r):
            pltpu.sync_copy(sh_flat.at[pl.ds(r * NBINS, NBINS)], mhist_v)

    with jax.named_scope("pass0"):
        @plsc.parallel_loop(0, VPC, unroll=4)
        def _(j):
            b = lax.bitcast_convert_type(x_v[pl.ds(j * L, L)], jnp.uint32)
            neg = (b >> 31) == jnp.uint32(1)
            key = jnp.where(neg, ~b, b | jnp.uint32(0x80000000))
            key_v[pl.ds(j * L, L)] = key
            digit = (key >> jnp.uint32(24)).astype(jnp.int32)
            plsc.addupdate_scatter(hist_v, [digit], ones_i)

    merge_and_read(0)
    with jax.named_scope("scan0"):
        bstar, rem = select_digit(rem)
        pref = bstar.astype(jnp.uint32) << jnp.uint32(24)

    for r in (1, 2, 3):
        shift = 24 - 8 * r
        hi = jnp.uint32(32 - 8 * r)
        for t in range(NBV):
            hist_v[pl.ds(t * L, L)] = zeros_i
        ph = pref >> hi

        with jax.named_scope("passN"):
            @plsc.parallel_loop(0, VPC, unroll=4)
            def _(j, hi=hi, shift=shift, ph=ph):
                key = key_v[pl.ds(j * L, L)]
                active = (key >> hi) == ph
                digit = ((key >> jnp.uint32(shift)) & jnp.uint32(0xFF))
                plsc.addupdate_scatter(hist_v, [digit.astype(jnp.int32)],
                                       ones_i, mask=active)

        if r == 3:
            pltpu.sync_copy(hist_v,
                            sh_flat.at[pl.ds(ROWS + sid * NBINS, NBINS)])
        merge_and_read(r)
        with jax.named_scope("scan%d" % r):
            bstar, rem = select_digit(rem)
            pref = pref | (bstar.astype(jnp.uint32) << jnp.uint32(shift))

    thresh = pref

    eqi_v[...] = ROWS + iota * NBINS + bstar
    pltpu.sync_copy(sh_flat.at[eqi_v], eq_v)
    ecnt = eq_v[...]
    exclv = plsc.cumsum(ecnt) - ecnt
    ecarry = jnp.sum(jnp.where(iota == sid, exclv, 0))

    with jax.named_scope("final"):
        @plsc.parallel_loop(0, VPC, unroll=4, carry=zeros_i)
        def _(j, qcarry):
            key = key_v[pl.ds(j * L, L)]
            x = x_v[pl.ds(j * L, L)]
            eq = key == thresh
            eqi = eq.astype(jnp.int32)
            incl = plsc.cumsum(eqi)
            rank = incl - eqi + qcarry + ecarry
            keep = (key > thresh) | (eq & (rank < rem))
            out_v[pl.ds(j * L, L)] = jnp.where(keep, x, jnp.float32(0.0))
            return qcarry + plsc.all_reduce_population_count(eq)

    with jax.named_scope("outdma"):
        pltpu.sync_copy(out_v, out_hbm.at[pl.ds(base, CHUNK)])


@jax.jit
def _topk_mask_sc(x_flat):
    mesh = plsc.VectorSubcoreMesh(core_axis_name="c", subcore_axis_name="s",
                                  num_cores=1, num_subcores=NS)
    return pl.kernel(
        _sc_topk_body,
        out_type=jax.ShapeDtypeStruct((N,), jnp.float32),
        mesh=mesh,
        compiler_params=pltpu.CompilerParams(needs_layout_passes=False),
        scratch_types=[
            pltpu.VMEM((CHUNK,), jnp.float32),
            pltpu.VMEM((CHUNK,), jnp.uint32),
            pltpu.VMEM((NBINS,), jnp.int32),
            pltpu.VMEM((NBINS,), jnp.int32),
            pltpu.VMEM((8, 128), jnp.int32),
            pltpu.VMEM((L,), jnp.int32),
            pltpu.VMEM((L,), jnp.int32),
            pltpu.VMEM((CHUNK,), jnp.float32),
            pltpu.VMEM((128,), jnp.int32),
            pltpu.SemaphoreType.DMA,
            pltpu.SemaphoreType.DMA,
            pltpu.VMEM_SHARED((5120,), jnp.int32),
        ],
    )(x_flat)


def kernel(score_vector):
    return _topk_mask_sc(score_vector.reshape(N)).reshape(1, N)


# speedup vs baseline: 1.0684x; 1.0042x over previous
import jax
import jax.numpy as jnp
from jax import lax
from jax.experimental import pallas as pl
from jax.experimental.pallas import tpu as pltpu
from jax.experimental.pallas import tpu_sc as plsc

N = 32768
K = 25
NS = 16
L = 16
CHUNK = N // NS
VPC = CHUNK // L
NBINS = 256
NBV = NBINS // L
ROWS = 1024


def _sc_topk_body(x_hbm, out_hbm, x_v, key_v, hist_v, mhist_v,
                  idx_v, eqi_v, eq_v, out_v, tmp_v, sem, sem2, sh_flat):
    sid = lax.axis_index("s")
    base = sid * CHUNK
    cp = pltpu.async_copy(x_hbm.at[pl.ds(base, CHUNK)], x_v, sem)

    iota = lax.iota(jnp.int32, L)
    zeros_i = jnp.zeros((L,), jnp.int32)
    ones_i = jnp.ones((L,), jnp.int32)

    for g in range(8):
        for t in range(8):
            idx_v[g, pl.ds(t * L, L)] = iota + (g * 128 + t * L)
    for t in range(4):
        tmp_v[pl.ds(t * L, L)] = zeros_i
    for t in range(NBV):
        hist_v[pl.ds(t * L, L)] = zeros_i
    pltpu.sync_copy(tmp_v.at[pl.ds(0, 64)], sh_flat.at[pl.ds(sid * 64, 64)])
    cp.wait()
    plsc.subcore_barrier()

    pref = jnp.zeros((L,), jnp.uint32)
    rem = jnp.full((L,), K, jnp.int32)

    def select_digit(rem):
        totals = zeros_i
        for t in range(NBV):
            h = mhist_v[pl.ds(t * L, L)]
            totals = totals + jnp.where(iota == t, jnp.sum(h), 0)
        rts = lax.rev(plsc.cumsum(lax.rev(totals, (0,))), (0,))
        tstar = plsc.all_reduce_population_count(rts >= rem) - 1
        excl = jnp.sum(jnp.where(iota == tstar, rts - totals, 0))
        ts = jnp.max(tstar)
        h = mhist_v[pl.ds(ts * L, L)]
        suf = lax.rev(plsc.cumsum(lax.rev(h, (0,))), (0,)) + excl
        cnt_in = plsc.all_reduce_population_count(suf >= rem)
        bstar = tstar * L + cnt_in - 1
        lane = cnt_in - 1
        suf_b = jnp.sum(jnp.where(iota == lane, suf, 0))
        h_b = jnp.sum(jnp.where(iota == lane, h, 0))
        return bstar, rem - (suf_b - h_b)

    def merge_and_read(r):
        c1 = pltpu.async_copy(hist_v.at[pl.ds(0, 128)],
                              sh_flat.at[idx_v.at[2 * r]], sem, add=True)
        c2 = pltpu.async_copy(hist_v.at[pl.ds(128, 128)],
                              sh_flat.at[idx_v.at[2 * r + 1]], sem2, add=True)
        c1.wait()
        c2.wait()
        plsc.subcore_barrier()
        pltpu.sync_copy(sh_flat.at[pl.ds(r * NBINS, NBINS)], mhist_v)

    @plsc.parallel_loop(0, VPC, unroll=8)
    def _(j):
        b = lax.bitcast_convert_type(x_v[pl.ds(j * L, L)], jnp.uint32)
        neg = (b >> 31) == jnp.uint32(1)
        key = jnp.where(neg, ~b, b | jnp.uint32(0x80000000))
        key_v[pl.ds(j * L, L)] = key
        digit = (key >> jnp.uint32(24)).astype(jnp.int32)
        plsc.addupdate_scatter(hist_v, [digit], ones_i)

    merge_and_read(0)
    bstar, rem = select_digit(rem)
    pref = bstar.astype(jnp.uint32) << jnp.uint32(24)

    for r in (1, 2, 3):
        shift = 24 - 8 * r
        hi = jnp.uint32(32 - 8 * r)
        for t in range(NBV):
            hist_v[pl.ds(t * L, L)] = zeros_i
        ph = pref >> hi

        @plsc.parallel_loop(0, VPC, unroll=8)
        def _(j, hi=hi, shift=shift, ph=ph):
            key = key_v[pl.ds(j * L, L)]
            active = (key >> hi) == ph
            digit = ((key >> jnp.uint32(shift)) & jnp.uint32(0xFF))
            plsc.addupdate_scatter(hist_v, [digit.astype(jnp.int32)],
                                   ones_i, mask=active)

        if r == 3:
            pltpu.sync_copy(hist_v,
                            sh_flat.at[pl.ds(ROWS + sid * NBINS, NBINS)])
        merge_and_read(r)
        bstar, rem = select_digit(rem)
        pref = pref | (bstar.astype(jnp.uint32) << jnp.uint32(shift))

    thresh = pref

    eqi_v[...] = ROWS + iota * NBINS + bstar
    pltpu.sync_copy(sh_flat.at[eqi_v], eq_v)
    ecnt = eq_v[...]
    exclv = plsc.cumsum(ecnt) - ecnt
    ecarry = jnp.sum(jnp.where(iota == sid, exclv, 0))

    @plsc.parallel_loop(0, VPC, unroll=8, carry=zeros_i)
    def _(j, qcarry):
        key = key_v[pl.ds(j * L, L)]
        x = x_v[pl.ds(j * L, L)]
        eq = key == thresh
        eqi = eq.astype(jnp.int32)
        incl = plsc.cumsum(eqi)
        rank = incl - eqi + qcarry + ecarry
        keep = (key > thresh) | (eq & (rank < rem))
        out_v[pl.ds(j * L, L)] = jnp.where(keep, x, jnp.float32(0.0))
        return qcarry + plsc.all_reduce_population_count(eq)

    pltpu.sync_copy(out_v, out_hbm.at[pl.ds(base, CHUNK)])


@jax.jit
def _topk_mask_sc(x_flat):
    mesh = plsc.VectorSubcoreMesh(core_axis_name="c", subcore_axis_name="s",
                                  num_cores=1, num_subcores=NS)
    return pl.kernel(
        _sc_topk_body,
        out_type=jax.ShapeDtypeStruct((N,), jnp.float32),
        mesh=mesh,
        compiler_params=pltpu.CompilerParams(needs_layout_passes=False),
        scratch_types=[
            pltpu.VMEM((CHUNK,), jnp.float32),
            pltpu.VMEM((CHUNK,), jnp.uint32),
            pltpu.VMEM((NBINS,), jnp.int32),
            pltpu.VMEM((NBINS,), jnp.int32),
            pltpu.VMEM((8, 128), jnp.int32),
            pltpu.VMEM((L,), jnp.int32),
            pltpu.VMEM((L,), jnp.int32),
            pltpu.VMEM((CHUNK,), jnp.float32),
            pltpu.VMEM((128,), jnp.int32),
            pltpu.SemaphoreType.DMA,
            pltpu.SemaphoreType.DMA,
            pltpu.VMEM_SHARED((5120,), jnp.int32),
        ],
    )(x_flat)


def kernel(score_vector):
    return _topk_mask_sc(score_vector.reshape(N)).reshape(1, N)
